# baseline (speedup 1.0000x reference)
import functools

import jax
import jax.numpy as jnp
from jax import lax
from jax.experimental import pallas as pl
from jax.experimental.pallas import tpu as pltpu
from jax.experimental.pallas import tpu_sc as plsc

B = 16384
DIM = 32
N_TAB = 1000000
N_USERS = 100000
F = 13
NF = 2 * F

NC, NS, L = 2, 16, 16
NW = NC * NS
SPT = B // NW
GROUP = 64
NGROUPS = SPT // GROUP
RPG = GROUP * NF
IDXN = RPG + L


def _rsqrt(n2):
    bits = lax.bitcast_convert_type(n2, jnp.int32)
    y = lax.bitcast_convert_type(jnp.int32(0x5F3759DF) - (bits >> 1), jnp.float32)
    h = 0.5 * n2
    for _ in range(2):
        y = y * (1.5 - h * y * y)
    return y


@functools.partial(
    pl.kernel,
    out_type=jax.ShapeDtypeStruct((B,), jnp.float32),
    mesh=plsc.VectorSubcoreMesh(core_axis_name="c", subcore_axis_name="s"),
    compiler_params=pltpu.CompilerParams(
        needs_layout_passes=False, use_tc_tiling_on_sc=False
    ),
    scratch_types=[
        pltpu.VMEM((SPT * 2,), jnp.int32),
        pltpu.VMEM((SPT * 2, 16), jnp.int32),
        pltpu.VMEM((IDXN,), jnp.int32),
        pltpu.VMEM((IDXN,), jnp.int32),
        pltpu.VMEM((2, RPG, DIM), jnp.float32),
        pltpu.VMEM((DIM, L), jnp.float32),
        pltpu.VMEM((SPT,), jnp.float32),
        pltpu.SemaphoreType.DMA,
        pltpu.SemaphoreType.DMA,
    ],
)
def _fm_sc(pair_hbm, ftab_hbm, tab_hbm, out_hbm,
           pairv, featv, id0, id1, emb, sacc, outv, sem0, sem1):
    wid = lax.axis_index("s") * NC + lax.axis_index("c")
    sems = (sem0, sem1)
    ids = (id0, id1)

    pltpu.sync_copy(pair_hbm.at[pl.ds(wid * (SPT * 2), SPT * 2)], pairv)
    pltpu.async_copy(ftab_hbm.at[pairv], featv, sem0).wait()

    def repack(g, b):
        ib = ids[b]
        for s in range(GROUP):
            r = g * (GROUP * 2) + 2 * s
            uv = _permute_ids(featv[r, :], DIM, TAB_BLK_ROWS)
            iv = _permute_ids(featv[r + 1, :], DIM, TAB_BLK_ROWS)
            ib[pl.ds(s * NF, L)] = uv
            ib[pl.ds(s * NF + F, L)] = iv

    def g2_start(b):
        pltpu.async_copy(
            tab_hbm.at[ids[b].at[pl.ds(0, RPG)]], emb.at[b], sems[b]
        )

    def g2_wait(b):
        pltpu.make_async_copy(
            tab_hbm.at[ids[b].at[pl.ds(0, RPG)]], emb.at[b], sems[b]
        ).wait()

    iota26 = lax.iota(jnp.int32, L) * NF
    zero = jnp.zeros((L,), jnp.float32)

    def chunk_compute(b, g, c):
        embref = emb.at[b]
        for d in range(DIM):
            sacc[d, :] = zero

        def f_body(f, ssq):
            rowv = iota26 + (c * (L * NF) + f)
            vs = []
            n2 = zero
            for d in range(DIM):
                colv = jnp.full((L,), d, jnp.int32)
                v = plsc.load_gather(embref, [rowv, colv])
                vs.append(v)
                n2 = n2 + v * v
            r = _rsqrt(n2)
            scale = jnp.where(n2 > 1.0, r, 1.0)
            for d in range(DIM):
                plsc.addupdate(sacc.at[d], scale * vs[d])
            return ssq + scale * scale * n2

        ssq = lax.fori_loop(0, NF, f_body, zero, unroll=2)
        acc = zero
        for d in range(DIM):
            sd = sacc[d, :]
            acc = acc + sd * sd
        out = 0.5 * (acc - ssq)
        logit = 1.0 / (1.0 + jnp.exp(-out))
        outv[pl.ds(g * GROUP + c * L, L)] = logit

    repack(0, 0)
    g2_start(0)
    repack(1, 1)
    g2_start(1)

    def two_groups(t, carry):
        for b in (0, 1):
            g = t * 2 + b
            g2_wait(b)
            for c in range(GROUP // L):
                chunk_compute(b, g, c)

            @pl.when(g + 2 < NGROUPS)
            def _():
                repack(g + 2, b)
                g2_start(b)

        return carry

    lax.fori_loop(0, NGROUPS // 2, two_groups, 0)
    pltpu.sync_copy(outv, out_hbm.at[pl.ds(wid * SPT, SPT)])


TAB_BLK_ROWS = 8192
FTAB_BLK_ROWS = 1024


def _permute_ids(r, width, blk_rows):
    k = 128 // width
    bc = blk_rows * k
    kb = k.bit_length() - 1
    rb = blk_rows.bit_length() - 1
    return ((r >> (rb + kb)) << (rb + kb)) | ((r & (blk_rows - 1)) << kb) \
        | ((r >> rb) & (k - 1))


def _to_linear_rows(x_t, rows, width, blk_rows):
    blk_cols = blk_rows * 128 // width
    grid = (rows + blk_cols - 1) // blk_cols
    fdt = x_t.dtype == jnp.float32

    k = 128 // width

    def body(xb, ob):
        eye = (lax.broadcasted_iota(jnp.int32, (width, width), 0)
               == lax.broadcasted_iota(jnp.int32, (width, width), 1)
               ).astype(jnp.float32)
        x = xb[...]
        x = x if fdt else lax.bitcast_convert_type(x, jnp.float32)
        t = x.T
        o = jnp.concatenate(
            [lax.slice(t, (m * blk_rows, 0), ((m + 1) * blk_rows, width))
             for m in range(k)],
            axis=1,
        )
        ob[...] = o if fdt else lax.bitcast_convert_type(o, jnp.int32)

    out128 = pl.pallas_call(
        body,
        grid=(grid,),
        in_specs=[pl.BlockSpec((width, blk_cols), lambda i: (0, i))],
        out_specs=pl.BlockSpec((blk_rows, 128), lambda i: (i, 0)),
        out_shape=jax.ShapeDtypeStruct((grid * blk_rows, 128), x_t.dtype),
    )(x_t)
    return out128.reshape(-1).reshape(grid * blk_cols, width)


def kernel(u, i, user_df, item_df, table):
    u = u.astype(jnp.int32)
    i = i.astype(jnp.int32)
    ftab_t = jnp.concatenate(
        [user_df.astype(jnp.int32).T, item_df.astype(jnp.int32).T], axis=1
    )
    ftab_t = jnp.pad(ftab_t, ((0, 16 - F), (0, 0)))
    ftab = _to_linear_rows(ftab_t, 2 * N_USERS, 16, FTAB_BLK_ROWS)
    tab = _to_linear_rows(table.T, N_TAB, DIM, TAB_BLK_ROWS)
    pair = _permute_ids(
        jnp.stack([u, i + N_USERS], axis=1).reshape(B * 2), 16, FTAB_BLK_ROWS
    )
    return _fm_sc(pair, ftab, tab)


# speedup vs baseline: 1.0050x; 1.0050x over previous
import functools

import jax
import jax.numpy as jnp
from jax import lax
from jax.experimental import pallas as pl
from jax.experimental.pallas import tpu as pltpu
from jax.experimental.pallas import tpu_sc as plsc

B = 16384
DIM = 32
N_TAB = 1000000
N_USERS = 100000
F = 13
NF = 2 * F

NC, NS, L = 2, 16, 16
NW = NC * NS
SPT = B // NW
GROUP = 32
NGROUPS = SPT // GROUP
RPG = GROUP * NF
IDXN = RPG + L


def _rsqrt(n2):
    bits = lax.bitcast_convert_type(n2, jnp.int32)
    y = lax.bitcast_convert_type(jnp.int32(0x5F3759DF) - (bits >> 1), jnp.float32)
    h = 0.5 * n2
    for _ in range(2):
        y = y * (1.5 - h * y * y)
    return y


@functools.partial(
    pl.kernel,
    out_type=jax.ShapeDtypeStruct((B,), jnp.float32),
    mesh=plsc.VectorSubcoreMesh(core_axis_name="c", subcore_axis_name="s"),
    compiler_params=pltpu.CompilerParams(
        needs_layout_passes=False, use_tc_tiling_on_sc=False
    ),
    scratch_types=[
        pltpu.VMEM((SPT * 2,), jnp.int32),
        pltpu.VMEM((SPT * 2, 16), jnp.int32),
        pltpu.VMEM((IDXN,), jnp.int32),
        pltpu.VMEM((IDXN,), jnp.int32),
        pltpu.VMEM((IDXN,), jnp.int32),
        pltpu.VMEM((3, RPG, DIM), jnp.float32),
        pltpu.VMEM((DIM, L), jnp.float32),
        pltpu.VMEM((SPT,), jnp.float32),
        pltpu.SemaphoreType.DMA,
        pltpu.SemaphoreType.DMA,
        pltpu.SemaphoreType.DMA,
    ],
)
def _fm_sc(pair_hbm, ftab_hbm, tab_hbm, out_hbm,
           pairv, featv, id0, id1, id2, emb, sacc, outv, sem0, sem1, sem2):
    wid = lax.axis_index("s") * NC + lax.axis_index("c")
    sems = (sem0, sem1, sem2)
    ids = (id0, id1, id2)

    pltpu.sync_copy(pair_hbm.at[pl.ds(wid * (SPT * 2), SPT * 2)], pairv)
    pltpu.async_copy(ftab_hbm.at[pairv], featv, sem0).wait()

    def repack(g, b):
        ib = ids[b]
        for s in range(GROUP):
            r = g * (GROUP * 2) + 2 * s
            uv = _permute_ids(featv[r, :], DIM, TAB_BLK_ROWS)
            iv = _permute_ids(featv[r + 1, :], DIM, TAB_BLK_ROWS)
            ib[pl.ds(s * NF, L)] = uv
            ib[pl.ds(s * NF + F, L)] = iv

    def g2_start(b):
        pltpu.async_copy(
            tab_hbm.at[ids[b].at[pl.ds(0, RPG)]], emb.at[b], sems[b]
        )

    def g2_wait(b):
        pltpu.make_async_copy(
            tab_hbm.at[ids[b].at[pl.ds(0, RPG)]], emb.at[b], sems[b]
        ).wait()

    iota26 = lax.iota(jnp.int32, L) * NF
    zero = jnp.zeros((L,), jnp.float32)

    def chunk_compute(b, g, c):
        embref = emb.at[b]
        for d in range(DIM):
            sacc[d, :] = zero

        def f_body(f, ssq):
            rowv = iota26 + (c * (L * NF) + f)
            vs = []
            n2 = zero
            for d in range(DIM):
                colv = jnp.full((L,), d, jnp.int32)
                v = plsc.load_gather(embref, [rowv, colv])
                vs.append(v)
                n2 = n2 + v * v
            r = _rsqrt(n2)
            scale = jnp.where(n2 > 1.0, r, 1.0)
            for d in range(DIM):
                plsc.addupdate(sacc.at[d], scale * vs[d])
            return ssq + scale * scale * n2

        ssq = lax.fori_loop(0, NF, f_body, zero, unroll=2)
        acc = zero
        for d in range(DIM):
            sd = sacc[d, :]
            acc = acc + sd * sd
        out = 0.5 * (acc - ssq)
        logit = 1.0 / (1.0 + jnp.exp(-out))
        outv[pl.ds(g * GROUP + c * L, L)] = logit

    repack(0, 0)
    g2_start(0)
    repack(1, 1)
    g2_start(1)

    def three_groups(t, carry):
        for j in (0, 1, 2):
            g = t * 3 + j
            g2_wait(j)

            @pl.when(g + 2 < NGROUPS)
            def _():
                b2 = (j + 2) % 3
                repack(g + 2, b2)
                g2_start(b2)

            for c in range(GROUP // L):
                chunk_compute(j, g, c)

        return carry

    lax.fori_loop(0, NGROUPS // 3, three_groups, 0)
    g_last = NGROUPS - 1
    b_last = g_last % 3
    g2_wait(b_last)
    for c in range(GROUP // L):
        chunk_compute(b_last, g_last, c)
    pltpu.sync_copy(outv, out_hbm.at[pl.ds(wid * SPT, SPT)])


TAB_BLK_ROWS = 8192
FTAB_BLK_ROWS = 1024


def _permute_ids(r, width, blk_rows):
    k = 128 // width
    bc = blk_rows * k
    kb = k.bit_length() - 1
    rb = blk_rows.bit_length() - 1
    return ((r >> (rb + kb)) << (rb + kb)) | ((r & (blk_rows - 1)) << kb) \
        | ((r >> rb) & (k - 1))


def _to_linear_rows(x_t, rows, width, blk_rows):
    blk_cols = blk_rows * 128 // width
    grid = (rows + blk_cols - 1) // blk_cols
    fdt = x_t.dtype == jnp.float32

    k = 128 // width

    def body(xb, ob):
        eye = (lax.broadcasted_iota(jnp.int32, (width, width), 0)
               == lax.broadcasted_iota(jnp.int32, (width, width), 1)
               ).astype(jnp.float32)
        x = xb[...]
        x = x if fdt else lax.bitcast_convert_type(x, jnp.float32)
        t = x.T
        o = jnp.concatenate(
            [lax.slice(t, (m * blk_rows, 0), ((m + 1) * blk_rows, width))
             for m in range(k)],
            axis=1,
        )
        ob[...] = o if fdt else lax.bitcast_convert_type(o, jnp.int32)

    out128 = pl.pallas_call(
        body,
        grid=(grid,),
        in_specs=[pl.BlockSpec((width, blk_cols), lambda i: (0, i))],
        out_specs=pl.BlockSpec((blk_rows, 128), lambda i: (i, 0)),
        out_shape=jax.ShapeDtypeStruct((grid * blk_rows, 128), x_t.dtype),
    )(x_t)
    return out128.reshape(-1).reshape(grid * blk_cols, width)


def kernel(u, i, user_df, item_df, table):
    u = u.astype(jnp.int32)
    i = i.astype(jnp.int32)
    ftab_t = jnp.concatenate(
        [user_df.astype(jnp.int32).T, item_df.astype(jnp.int32).T], axis=1
    )
    ftab_t = jnp.pad(ftab_t, ((0, 16 - F), (0, 0)))
    ftab = _to_linear_rows(ftab_t, 2 * N_USERS, 16, FTAB_BLK_ROWS)
    tab = _to_linear_rows(table.T, N_TAB, DIM, TAB_BLK_ROWS)
    pair = _permute_ids(
        jnp.stack([u, i + N_USERS], axis=1).reshape(B * 2), 16, FTAB_BLK_ROWS
    )
    return _fm_sc(pair, ftab, tab)
